# Initial kernel scaffold; baseline (speedup 1.0000x reference)
import jax
import jax.numpy as jnp
from jax.experimental import pallas as pl


def kernel(x, edge_index, pool_p, W0, gru_w_ih, gru_w_hh, gru_b_ih, gru_b_hh, lin_w, lin_b):
    raise NotImplementedError("write your pallas kernel here")


# speedup vs baseline: 17.4899x; 17.4899x over previous
import jax
import jax.numpy as jnp
from jax import lax
from jax.experimental import pallas as pl
from jax.experimental.pallas import tpu as pltpu
from jax.experimental.pallas import tpu_sc as plsc

N = 10000
F = 128
L = 64
E = 320000

NC = 2
NS = 16
NW = NC * NS

NPAD = 10240
ROWS_T = NPAD // NS
EDGES_W = E // NW
CHUNK = 80
NCHUNK = EDGES_W // CHUNK
SROWS = NPAD // 128

NEG = -3.0e38


def _score_kernel(x_ref, p_ref, s_ref):
    p = p_ref[...]
    pnorm = jnp.sqrt(jnp.sum(p * p))
    xb = x_ref[...].astype(jnp.bfloat16).astype(jnp.float32)
    pb = p.astype(jnp.bfloat16).astype(jnp.float32)
    s = lax.dot_general(xb, pb, (((1,), (1,)), ((), ())),
                        preferred_element_type=jnp.float32)
    s_ref[...] = s / pnorm


def _score(x, pool_p):
    return pl.pallas_call(
        _score_kernel,
        out_shape=jax.ShapeDtypeStruct((N, 1), jnp.float32),
    )(x, pool_p.reshape(1, F))


def _evolve_kernel(s_ref, x_ref, w0_ref, wih_ref, whh_ref, bih_ref,
                   bhh_ref, w_out_ref, score_ref, perm_ref, topv_ref,
                   xt_ref):
    score_ref[...] = s_ref[...]
    idx2 = (lax.broadcasted_iota(jnp.int32, (SROWS, 128), 0) * 128
            + lax.broadcasted_iota(jnp.int32, (SROWS, 128), 1))

    def topk_body(i, _):
        sc = score_ref[...]
        m = jnp.max(sc)
        am = jnp.min(jnp.where(sc == m, idx2, jnp.int32(2 ** 30)))
        perm_ref[i] = am
        topv_ref[i] = m
        score_ref[...] = jnp.where(idx2 == am, NEG, sc)
        return 0

    lax.fori_loop(0, F, topk_body, 0)

    def gather_body(i, _):
        pi = perm_ref[i]
        tv = topv_ref[i]
        row = x_ref[pl.ds(pi, 1), :]
        xt_ref[pl.ds(i, 1), :] = row * jnp.tanh(
            jnp.broadcast_to(tv, (1, F)))
        return 0

    lax.fori_loop(0, F, gather_body, 0)

    xt = xt_ref[...]
    w0 = w0_ref[...]
    gi = lax.dot_general(xt, wih_ref[...], (((1,), (1,)), ((), ())),
                         preferred_element_type=jnp.float32) + bih_ref[...]
    gh = lax.dot_general(w0, whh_ref[...], (((1,), (1,)), ((), ())),
                         preferred_element_type=jnp.float32) + bhh_ref[...]
    i_r, i_z, i_n = gi[:, :F], gi[:, F:2 * F], gi[:, 2 * F:]
    h_r, h_z, h_n = gh[:, :F], gh[:, F:2 * F], gh[:, 2 * F:]
    r = jax.nn.sigmoid(i_r + h_r)
    z = jax.nn.sigmoid(i_z + h_z)
    n = jnp.tanh(i_n + r * h_n)
    w_out_ref[...] = (1.0 - z) * n + z * w0


def _evolve_w(score2d, x, w0, wih, whh, bih, bhh):
    return pl.pallas_call(
        _evolve_kernel,
        out_shape=jax.ShapeDtypeStruct((F, F), jnp.float32),
        scratch_shapes=[
            pltpu.VMEM((SROWS, 128), jnp.float32),
            pltpu.SMEM((F,), jnp.int32),
            pltpu.SMEM((F,), jnp.float32),
            pltpu.VMEM((F, F), jnp.float32),
        ],
    )(score2d, x, w0, wih, whh, bih.reshape(1, 3 * F),
      bhh.reshape(1, 3 * F))


def _deg_kernel(dst_hbm, deg_out, idx_v, ones_v, zero_v, deg_sh):
    c = lax.axis_index("c")
    s = lax.axis_index("s")
    wid = c * NS + s
    zv = jnp.zeros((16,), jnp.float32)
    for q in range(ROWS_T // 16):
        zero_v[pl.ds(q * 16, 16)] = zv
    ov = jnp.ones((16,), jnp.float32)
    for q in range(CHUNK // 16):
        ones_v[pl.ds(q * 16, 16)] = ov
    pltpu.sync_copy(zero_v, deg_sh.at[pl.ds(s * ROWS_T, ROWS_T)])
    plsc.subcore_barrier()

    def body(j, _):
        base = wid * EDGES_W + j * CHUNK
        pltpu.sync_copy(dst_hbm.at[pl.ds(base, CHUNK)], idx_v)
        pltpu.sync_copy(ones_v, deg_sh.at[idx_v], add=True)
        return 0

    lax.fori_loop(0, NCHUNK, body, 0)
    plsc.subcore_barrier()
    pltpu.sync_copy(deg_sh.at[pl.ds(s * ROWS_T, ROWS_T)], zero_v)
    pltpu.sync_copy(zero_v,
                    deg_out.at[pl.ds(c * NPAD + s * ROWS_T, ROWS_T)])


def _degrees(dst):
    k = pl.kernel(
        _deg_kernel,
        out_type=jax.ShapeDtypeStruct((NC * NPAD,), jnp.float32),
        mesh=plsc.VectorSubcoreMesh(core_axis_name="c",
                                    subcore_axis_name="s"),
        scratch_types=[
            pltpu.VMEM((CHUNK,), jnp.int32),
            pltpu.VMEM((CHUNK,), jnp.float32),
            pltpu.VMEM((ROWS_T,), jnp.float32),
            pltpu.VMEM_SHARED((NPAD,), jnp.float32),
        ],
    )
    return k(dst)


def _xw_kernel(x_ref, w_ref, d0_ref, d1_ref, y_ref, dinv_ref):
    deg = d0_ref[...] + d1_ref[...] + 1.0
    dinv = lax.rsqrt(deg)
    xw = jnp.dot(x_ref[...], w_ref[...],
                 preferred_element_type=jnp.float32)
    y_ref[...] = xw * dinv
    dinv_ref[...] = dinv


def _compute_y(x, w, d0, d1):
    blk = 2048
    grid = (N + blk - 1) // blk
    return pl.pallas_call(
        _xw_kernel,
        grid=(grid,),
        out_shape=(jax.ShapeDtypeStruct((N, F), jnp.float32),
                   jax.ShapeDtypeStruct((N, 1), jnp.float32)),
        in_specs=[
            pl.BlockSpec((blk, F), lambda i: (i, 0)),
            pl.BlockSpec((F, F), lambda i: (0, 0)),
            pl.BlockSpec((blk, 1), lambda i: (i, 0)),
            pl.BlockSpec((blk, 1), lambda i: (i, 0)),
        ],
        out_specs=(pl.BlockSpec((blk, F), lambda i: (i, 0)),
                   pl.BlockSpec((blk, 1), lambda i: (i, 0))),
    )(x, w, d0, d1)


def _agg_kernel(src_hbm, dst_hbm, y_hbm, part_out,
                sidx_v, didx_v, rows_v, zrows_v, sem, acc_sh):
    c = lax.axis_index("c")
    s = lax.axis_index("s")
    wid = c * NS + s
    zv = jnp.zeros((16,), jnp.float32)
    for rr in range(ROWS_T // 8):
        for cc in range(F // 16):
            zrows_v[rr, pl.ds(cc * 16, 16)] = zv
    for q in range(8):
        pltpu.sync_copy(
            zrows_v,
            acc_sh.at[pl.ds(s * ROWS_T + q * (ROWS_T // 8), ROWS_T // 8)])
    plsc.subcore_barrier()

    def body(j, _):
        base = wid * EDGES_W + j * CHUNK
        pltpu.sync_copy(src_hbm.at[pl.ds(base, CHUNK)], sidx_v)
        pltpu.sync_copy(dst_hbm.at[pl.ds(base, CHUNK)], didx_v)
        pltpu.async_copy(y_hbm.at[sidx_v], rows_v, sem).wait()
        pltpu.sync_copy(rows_v, acc_sh.at[didx_v], add=True)
        return 0

    lax.fori_loop(0, NCHUNK, body, 0)
    plsc.subcore_barrier()
    for q in range(8):
        off = s * ROWS_T + q * (ROWS_T // 8)
        pltpu.sync_copy(acc_sh.at[pl.ds(off, ROWS_T // 8)], zrows_v)
        pltpu.sync_copy(zrows_v,
                        part_out.at[pl.ds(c * NPAD + off, ROWS_T // 8)])


def _aggregate(src, dst, y):
    k = pl.kernel(
        _agg_kernel,
        out_type=jax.ShapeDtypeStruct((NC * NPAD, F), jnp.float32),
        mesh=plsc.VectorSubcoreMesh(core_axis_name="c",
                                    subcore_axis_name="s"),
        scratch_types=[
            pltpu.VMEM((CHUNK,), jnp.int32),
            pltpu.VMEM((CHUNK,), jnp.int32),
            pltpu.VMEM((CHUNK, F), jnp.float32),
            pltpu.VMEM((ROWS_T // 8, F), jnp.float32),
            pltpu.SemaphoreType.DMA,
            pltpu.VMEM_SHARED((NPAD, F), jnp.float32),
        ],
    )
    return k(src, dst, y)


def _head_kernel(p_ref, y_ref, dinv_ref, lw_ref, lb_ref, out_ref):
    t = p_ref[0] + p_ref[1] + y_ref[...]
    h = jnp.maximum(t, 0.0) * dinv_ref[...]
    out_ref[...] = lax.dot_general(
        h, lw_ref[...], (((1,), (1,)), ((), ())),
        preferred_element_type=jnp.float32) + lb_ref[...]


def _head(parts, y, dinv, lin_w, lin_b):
    blk = 2048
    grid = NPAD // blk
    return pl.pallas_call(
        _head_kernel,
        grid=(grid,),
        out_shape=jax.ShapeDtypeStruct((N, L), jnp.float32),
        in_specs=[
            pl.BlockSpec((NC, blk, F), lambda i: (0, i, 0)),
            pl.BlockSpec((blk, F), lambda i: (i, 0)),
            pl.BlockSpec((blk, 1), lambda i: (i, 0)),
            pl.BlockSpec((L, F), lambda i: (0, 0)),
            pl.BlockSpec((1, L), lambda i: (0, 0)),
        ],
        out_specs=pl.BlockSpec((blk, L), lambda i: (i, 0)),
    )(parts, y, dinv, lin_w, lin_b.reshape(1, L))


def kernel(x, edge_index, pool_p, W0, gru_w_ih, gru_w_hh, gru_b_ih,
           gru_b_hh, lin_w, lin_b):
    src = edge_index[0]
    dst = edge_index[1]
    s_col = _score(x, pool_p)
    score2d = jnp.pad(s_col[:, 0], (0, NPAD - N),
                      constant_values=NEG).reshape(SROWS, 128)
    w = _evolve_w(score2d, x, W0, gru_w_ih, gru_w_hh, gru_b_ih, gru_b_hh)
    deg2 = _degrees(dst)
    d0 = deg2[:N].reshape(N, 1)
    d1 = deg2[NPAD:NPAD + N].reshape(N, 1)
    y, dinv = _compute_y(x, w, d0, d1)
    parts = _aggregate(src, dst, y).reshape(NC, NPAD, F)
    return _head(parts, y, dinv, lin_w, lin_b)


# speedup vs baseline: 29.2409x; 1.6719x over previous
import jax
import jax.numpy as jnp
from jax import lax
from jax.experimental import pallas as pl
from jax.experimental.pallas import tpu as pltpu
from jax.experimental.pallas import tpu_sc as plsc

N = 10000
F = 128
L = 64
E = 320000

NC = 2
NS = 16
NW = NC * NS

NPAD = 10240
ROWS_T = NPAD // NS
EDGES_W = E // NW
CHUNK = 80
NCHUNK = EDGES_W // CHUNK
C4 = 40
NCH4 = EDGES_W // C4
M4 = 4
NB4 = 2
MI4 = 8
SROWS = NPAD // 128
ZROWS = 40

NEG = -3.0e38


def _score_kernel(x_ref, p_ref, s_ref):
    p = p_ref[...]
    pnorm = jnp.sqrt(jnp.sum(p * p))
    xb = x_ref[...].astype(jnp.bfloat16).astype(jnp.float32)
    pb = p.astype(jnp.bfloat16).astype(jnp.float32)
    s = lax.dot_general(xb, pb, (((1,), (1,)), ((), ())),
                        preferred_element_type=jnp.float32)
    s_ref[...] = s / pnorm


def _score(x, pool_p):
    return pl.pallas_call(
        _score_kernel,
        out_shape=jax.ShapeDtypeStruct((N, 1), jnp.float32),
    )(x, pool_p.reshape(1, F))


def _evolve_kernel(s_ref, x_ref, w0_ref, wih_ref, whh_ref, bih_ref,
                   bhh_ref, w_out_ref, score_ref, perm_ref, topv_ref,
                   xt_ref):
    score_ref[...] = s_ref[...]
    idx2 = (lax.broadcasted_iota(jnp.int32, (SROWS, 128), 0) * 128
            + lax.broadcasted_iota(jnp.int32, (SROWS, 128), 1))

    def topk_body(i, _):
        sc = score_ref[...]
        m = jnp.max(sc)
        am = jnp.min(jnp.where(sc == m, idx2, jnp.int32(2 ** 30)))
        perm_ref[i] = am
        topv_ref[i] = m
        score_ref[...] = jnp.where(idx2 == am, NEG, sc)
        return 0

    lax.fori_loop(0, F, topk_body, 0)

    def gather_body(i, _):
        pi = perm_ref[i]
        tv = topv_ref[i]
        row = x_ref[pl.ds(pi, 1), :]
        xt_ref[pl.ds(i, 1), :] = row * jnp.tanh(
            jnp.broadcast_to(tv, (1, F)))
        return 0

    lax.fori_loop(0, F, gather_body, 0)

    xt = xt_ref[...]
    w0 = w0_ref[...]
    gi = lax.dot_general(xt, wih_ref[...], (((1,), (1,)), ((), ())),
                         preferred_element_type=jnp.float32) + bih_ref[...]
    gh = lax.dot_general(w0, whh_ref[...], (((1,), (1,)), ((), ())),
                         preferred_element_type=jnp.float32) + bhh_ref[...]
    i_r, i_z, i_n = gi[:, :F], gi[:, F:2 * F], gi[:, 2 * F:]
    h_r, h_z, h_n = gh[:, :F], gh[:, F:2 * F], gh[:, 2 * F:]
    r = jax.nn.sigmoid(i_r + h_r)
    z = jax.nn.sigmoid(i_z + h_z)
    n = jnp.tanh(i_n + r * h_n)
    w_out_ref[...] = (1.0 - z) * n + z * w0


def _evolve_w(score2d, x, w0, wih, whh, bih, bhh):
    return pl.pallas_call(
        _evolve_kernel,
        out_shape=jax.ShapeDtypeStruct((F, F), jnp.float32),
        scratch_shapes=[
            pltpu.VMEM((SROWS, 128), jnp.float32),
            pltpu.SMEM((F,), jnp.int32),
            pltpu.SMEM((F,), jnp.float32),
            pltpu.VMEM((F, F), jnp.float32),
        ],
    )(score2d, x, w0, wih, whh, bih.reshape(1, 3 * F),
      bhh.reshape(1, 3 * F))


def _deg_kernel(dst2_hbm, deg_out, didx_all, ones_v, zero_v,
                s0, s1, s2, s3, deg_sh):
    ssem = [s0, s1, s2, s3]
    c = lax.axis_index("c")
    s = lax.axis_index("s")
    wid = c * NS + s
    pltpu.sync_copy(dst2_hbm.at[wid], didx_all)
    zv = jnp.zeros((16,), jnp.float32)
    for q in range(ROWS_T // 16):
        zero_v[pl.ds(q * 16, 16)] = zv
    ov = jnp.ones((16,), jnp.float32)
    for q in range(CHUNK // 16):
        ones_v[pl.ds(q * 16, 16)] = ov
    pltpu.sync_copy(zero_v, deg_sh.at[pl.ds(s * ROWS_T, ROWS_T)])
    plsc.subcore_barrier()

    def visit(g, _):
        for b in range(4):
            j = g * 4 + b

            @pl.when(jnp.logical_and(j >= 4, j < NCHUNK))
            def _():
                pltpu.make_async_copy(ones_v, deg_sh.at[didx_all.at[0]],
                                      ssem[b]).wait()

            @pl.when(j < NCHUNK)
            def _():
                pltpu.async_copy(ones_v, deg_sh.at[didx_all.at[j]],
                                 ssem[b], add=True)
        return 0

    lax.fori_loop(0, (NCHUNK + 3) // 4, visit, 0)
    for b in range(4):
        pltpu.make_async_copy(ones_v, deg_sh.at[didx_all.at[0]],
                              ssem[b]).wait()
    plsc.subcore_barrier()
    pltpu.sync_copy(deg_sh.at[pl.ds(s * ROWS_T, ROWS_T)], zero_v)
    pltpu.sync_copy(zero_v,
                    deg_out.at[pl.ds(c * NPAD + s * ROWS_T, ROWS_T)])


def _degrees(dst):
    k = pl.kernel(
        _deg_kernel,
        out_type=jax.ShapeDtypeStruct((NC * NPAD,), jnp.float32),
        mesh=plsc.VectorSubcoreMesh(core_axis_name="c",
                                    subcore_axis_name="s"),
        scratch_types=(
            [pltpu.VMEM((NCHUNK, CHUNK), jnp.int32),
             pltpu.VMEM((CHUNK,), jnp.float32),
             pltpu.VMEM((ROWS_T,), jnp.float32)]
            + [pltpu.SemaphoreType.DMA] * 4
            + [pltpu.VMEM_SHARED((NPAD,), jnp.float32)]
        ),
    )
    return k(dst.reshape(NW, NCHUNK, CHUNK))


def _xw_kernel(x_ref, w_ref, d0_ref, d1_ref, y_ref, dinv_ref):
    deg = d0_ref[...] + d1_ref[...] + 1.0
    dinv = lax.rsqrt(deg)
    xw = jnp.dot(x_ref[...], w_ref[...],
                 preferred_element_type=jnp.float32)
    y_ref[...] = xw * dinv
    dinv_ref[...] = dinv


def _compute_y(x, w, d0, d1):
    blk = 2048
    grid = (N + blk - 1) // blk
    return pl.pallas_call(
        _xw_kernel,
        grid=(grid,),
        out_shape=(jax.ShapeDtypeStruct((N, F), jnp.float32),
                   jax.ShapeDtypeStruct((N, 1), jnp.float32)),
        in_specs=[
            pl.BlockSpec((blk, F), lambda i: (i, 0)),
            pl.BlockSpec((F, F), lambda i: (0, 0)),
            pl.BlockSpec((blk, 1), lambda i: (i, 0)),
            pl.BlockSpec((blk, 1), lambda i: (i, 0)),
        ],
        out_specs=(pl.BlockSpec((blk, F), lambda i: (i, 0)),
                   pl.BlockSpec((blk, 1), lambda i: (i, 0))),
    )(x, w, d0, d1)


def _agg_kernel(src_hbm, dst_hbm, y_hbm, part_out,
                r0, r1, r2, r3,
                g0, g1, g2, g3, s0, s1, s2, s3,
                si0, si1, si2, si3, si4, si5, si6, si7,
                di0, di1, di2, di3, di4, di5, di6, di7,
                i0, i1, i2, i3, i4, i5, i6, i7,
                zr0, zr1, o0, o1, acc_sh):
    rows = [r0, r1, r2, r3]
    gsem = [g0, g1, g2, g3]
    ssem = [s0, s1, s2, s3]
    sidx = [si0, si1, si2, si3, si4, si5, si6, si7]
    didx = [di0, di1, di2, di3, di4, di5, di6, di7]
    isem = [i0, i1, i2, i3, i4, i5, i6, i7]
    zr = [zr0, zr1]
    osem = [o0, o1]
    c = lax.axis_index("c")
    s = lax.axis_index("s")
    wid = c * NS + s
    ebase = wid * EDGES_W

    def idx_fire(k, slot):
        pltpu.async_copy(src_hbm.at[pl.ds(ebase + k * C4, C4)],
                         sidx[slot], isem[slot])
        pltpu.async_copy(dst_hbm.at[pl.ds(ebase + k * C4, C4)],
                         didx[slot], isem[slot])

    def idx_wait(k, slot):
        pltpu.make_async_copy(src_hbm.at[pl.ds(ebase, C4)],
                              sidx[slot], isem[slot]).wait()
        pltpu.make_async_copy(dst_hbm.at[pl.ds(ebase, C4)],
                              didx[slot], isem[slot]).wait()

    for k in range(2 * NB4):
        idx_fire(k, k)
    for b in range(NB4):
        idx_wait(b, b)
        pltpu.async_copy(y_hbm.at[sidx[b]], rows[b], gsem[b])

    zv = jnp.zeros((16,), jnp.float32)
    for rr in range(ZROWS):
        for cc in range(F // 16):
            zr0[rr, pl.ds(cc * 16, 16)] = zv
    for q in range(ROWS_T // ZROWS):
        pltpu.async_copy(
            zr0, acc_sh.at[pl.ds(s * ROWS_T + q * ZROWS, ZROWS)], o0)
    for q in range(ROWS_T // ZROWS):
        pltpu.make_async_copy(
            zr0, acc_sh.at[pl.ds(s * ROWS_T, ZROWS)], o0).wait()
    plsc.subcore_barrier()

    def group(g, _):
        for bb in range(MI4):
            b = bb % M4
            sf = (bb + NB4) % M4
            iu = (bb + NB4) % MI4
            ifi = (bb + 2 * NB4) % MI4
            j = g * MI4 + bb

            @pl.when(j < NCH4)
            def _():
                pltpu.make_async_copy(y_hbm.at[sidx[0]], rows[b],
                                      gsem[b]).wait()
                pltpu.async_copy(rows[b], acc_sh.at[didx[bb % MI4]],
                                 ssem[b], add=True)

            @pl.when(jnp.logical_and(j >= M4 - NB4, j + NB4 < NCH4))
            def _():
                pltpu.make_async_copy(rows[sf], acc_sh.at[didx[0]],
                                      ssem[sf]).wait()

            @pl.when(j + NB4 < NCH4)
            def _():
                idx_wait(j + NB4, iu)
                pltpu.async_copy(y_hbm.at[sidx[iu]], rows[sf], gsem[sf])

            @pl.when(j + 2 * NB4 < NCH4)
            def _():
                idx_fire(j + 2 * NB4, ifi)
        return 0

    lax.fori_loop(0, (NCH4 + MI4 - 1) // MI4, group, 0)
    for b in range(M4):
        pltpu.make_async_copy(rows[b], acc_sh.at[didx[0]],
                              ssem[b]).wait()
    plsc.subcore_barrier()

    NQ = ROWS_T // ZROWS
    for q in range(NQ):
        b = q % 2
        if q >= 2:
            pltpu.make_async_copy(
                zr[b], part_out.at[pl.ds(c * NPAD, ZROWS)], osem[b]).wait()
        off = s * ROWS_T + q * ZROWS
        pltpu.sync_copy(acc_sh.at[pl.ds(off, ZROWS)], zr[b])
        pltpu.async_copy(zr[b], part_out.at[pl.ds(c * NPAD + off, ZROWS)],
                         osem[b])
    for b in range(2):
        pltpu.make_async_copy(
            zr[b], part_out.at[pl.ds(c * NPAD, ZROWS)], osem[b]).wait()


def _aggregate(src, dst, y):
    k = pl.kernel(
        _agg_kernel,
        out_type=jax.ShapeDtypeStruct((NC * NPAD, F), jnp.float32),
        mesh=plsc.VectorSubcoreMesh(core_axis_name="c",
                                    subcore_axis_name="s"),
        scratch_types=(
            [pltpu.VMEM((C4, F), jnp.float32)] * 4
            + [pltpu.SemaphoreType.DMA] * 8
            + [pltpu.VMEM((C4,), jnp.int32)] * 16
            + [pltpu.SemaphoreType.DMA] * 8
            + [pltpu.VMEM((ZROWS, F), jnp.float32)] * 2
            + [pltpu.SemaphoreType.DMA] * 2
            + [pltpu.VMEM_SHARED((NPAD, F), jnp.float32)]
        ),
    )
    return k(src, dst, y)


def _head_kernel(p_ref, y_ref, dinv_ref, lw_ref, lb_ref, out_ref):
    t = p_ref[0] + p_ref[1] + y_ref[...]
    h = jnp.maximum(t, 0.0) * dinv_ref[...]
    out_ref[...] = lax.dot_general(
        h, lw_ref[...], (((1,), (1,)), ((), ())),
        preferred_element_type=jnp.float32) + lb_ref[...]


def _head(parts, y, dinv, lin_w, lin_b):
    blk = 2048
    grid = NPAD // blk
    return pl.pallas_call(
        _head_kernel,
        grid=(grid,),
        out_shape=jax.ShapeDtypeStruct((N, L), jnp.float32),
        in_specs=[
            pl.BlockSpec((NC, blk, F), lambda i: (0, i, 0)),
            pl.BlockSpec((blk, F), lambda i: (i, 0)),
            pl.BlockSpec((blk, 1), lambda i: (i, 0)),
            pl.BlockSpec((L, F), lambda i: (0, 0)),
            pl.BlockSpec((1, L), lambda i: (0, 0)),
        ],
        out_specs=pl.BlockSpec((blk, L), lambda i: (i, 0)),
    )(parts, y, dinv, lin_w, lin_b.reshape(1, L))


def kernel(x, edge_index, pool_p, W0, gru_w_ih, gru_w_hh, gru_b_ih,
           gru_b_hh, lin_w, lin_b):
    src = edge_index[0]
    dst = edge_index[1]
    s_col = _score(x, pool_p)
    score2d = jnp.pad(s_col[:, 0], (0, NPAD - N),
                      constant_values=NEG).reshape(SROWS, 128)
    w = _evolve_w(score2d, x, W0, gru_w_ih, gru_w_hh, gru_b_ih, gru_b_hh)
    deg2 = _degrees(dst)
    d0 = deg2[:N].reshape(N, 1)
    d1 = deg2[NPAD:NPAD + N].reshape(N, 1)
    y, dinv = _compute_y(x, w, d0, d1)
    parts = _aggregate(src, dst, y).reshape(NC, NPAD, F)
    return _head(parts, y, dinv, lin_w, lin_b)


# speedup vs baseline: 31.5681x; 1.0796x over previous
import jax
import jax.numpy as jnp
from jax import lax
from jax.experimental import pallas as pl
from jax.experimental.pallas import tpu as pltpu
from jax.experimental.pallas import tpu_sc as plsc

N = 10000
F = 128
L = 64
E = 320000

NC = 2
NS = 16
NW = NC * NS

NPAD = 10240
ROWS_T = NPAD // NS
EDGES_W = E // NW
CHUNK = 80
NCHUNK = EDGES_W // CHUNK
C4 = 40
NCH4 = EDGES_W // C4
M4 = 6
NB4 = 3
MI4 = 12
SROWS = NPAD // 128
ZROWS = 16

NEG = -3.0e38


def _score_kernel(x_ref, p_ref, s_ref):
    p = p_ref[...]
    pnorm = jnp.sqrt(jnp.sum(p * p))
    xb = x_ref[...].astype(jnp.bfloat16).astype(jnp.float32)
    pb = p.astype(jnp.bfloat16).astype(jnp.float32)
    s = lax.dot_general(xb, pb, (((1,), (1,)), ((), ())),
                        preferred_element_type=jnp.float32)
    s_ref[...] = s / pnorm


def _score(x, pool_p):
    return pl.pallas_call(
        _score_kernel,
        out_shape=jax.ShapeDtypeStruct((N, 1), jnp.float32),
    )(x, pool_p.reshape(1, F))


def _evolve_kernel(s_ref, x_ref, w0_ref, wih_ref, whh_ref, bih_ref,
                   bhh_ref, w_out_ref, score_ref, perm_ref, topv_ref,
                   xt_ref):
    score_ref[...] = s_ref[...]
    idx2 = (lax.broadcasted_iota(jnp.int32, (SROWS, 128), 0) * 128
            + lax.broadcasted_iota(jnp.int32, (SROWS, 128), 1))

    def topk_body(i, _):
        sc = score_ref[...]
        m = jnp.max(sc)
        am = jnp.min(jnp.where(sc == m, idx2, jnp.int32(2 ** 30)))
        perm_ref[i] = am
        topv_ref[i] = m
        score_ref[...] = jnp.where(idx2 == am, NEG, sc)
        return 0

    lax.fori_loop(0, F, topk_body, 0)

    def gather_body(i, _):
        pi = perm_ref[i]
        tv = topv_ref[i]
        row = x_ref[pl.ds(pi, 1), :]
        xt_ref[pl.ds(i, 1), :] = row * jnp.tanh(
            jnp.broadcast_to(tv, (1, F)))
        return 0

    lax.fori_loop(0, F, gather_body, 0)

    xt = xt_ref[...]
    w0 = w0_ref[...]
    gi = lax.dot_general(xt, wih_ref[...], (((1,), (1,)), ((), ())),
                         preferred_element_type=jnp.float32) + bih_ref[...]
    gh = lax.dot_general(w0, whh_ref[...], (((1,), (1,)), ((), ())),
                         preferred_element_type=jnp.float32) + bhh_ref[...]
    i_r, i_z, i_n = gi[:, :F], gi[:, F:2 * F], gi[:, 2 * F:]
    h_r, h_z, h_n = gh[:, :F], gh[:, F:2 * F], gh[:, 2 * F:]
    r = jax.nn.sigmoid(i_r + h_r)
    z = jax.nn.sigmoid(i_z + h_z)
    n = jnp.tanh(i_n + r * h_n)
    w_out_ref[...] = (1.0 - z) * n + z * w0


def _evolve_w(score2d, x, w0, wih, whh, bih, bhh):
    return pl.pallas_call(
        _evolve_kernel,
        out_shape=jax.ShapeDtypeStruct((F, F), jnp.float32),
        scratch_shapes=[
            pltpu.VMEM((SROWS, 128), jnp.float32),
            pltpu.SMEM((F,), jnp.int32),
            pltpu.SMEM((F,), jnp.float32),
            pltpu.VMEM((F, F), jnp.float32),
        ],
    )(score2d, x, w0, wih, whh, bih.reshape(1, 3 * F),
      bhh.reshape(1, 3 * F))


def _deg_kernel(dst2_hbm, deg_out, didx_all, ones_v, zero_v,
                s0, s1, s2, s3, deg_sh):
    ssem = [s0, s1, s2, s3]
    c = lax.axis_index("c")
    s = lax.axis_index("s")
    wid = c * NS + s
    pltpu.sync_copy(dst2_hbm.at[wid], didx_all)
    zv = jnp.zeros((16,), jnp.float32)
    for q in range(ROWS_T // 16):
        zero_v[pl.ds(q * 16, 16)] = zv
    ov = jnp.ones((16,), jnp.float32)
    for q in range(CHUNK // 16):
        ones_v[pl.ds(q * 16, 16)] = ov
    pltpu.sync_copy(zero_v, deg_sh.at[pl.ds(s * ROWS_T, ROWS_T)])
    plsc.subcore_barrier()

    def visit(g, _):
        for b in range(4):
            j = g * 4 + b

            @pl.when(jnp.logical_and(j >= 4, j < NCHUNK))
            def _():
                pltpu.make_async_copy(ones_v, deg_sh.at[didx_all.at[0]],
                                      ssem[b]).wait()

            @pl.when(j < NCHUNK)
            def _():
                pltpu.async_copy(ones_v, deg_sh.at[didx_all.at[j]],
                                 ssem[b], add=True)
        return 0

    lax.fori_loop(0, (NCHUNK + 3) // 4, visit, 0)
    for b in range(4):
        pltpu.make_async_copy(ones_v, deg_sh.at[didx_all.at[0]],
                              ssem[b]).wait()
    plsc.subcore_barrier()
    pltpu.sync_copy(deg_sh.at[pl.ds(s * ROWS_T, ROWS_T)], zero_v)
    pltpu.sync_copy(zero_v,
                    deg_out.at[pl.ds(c * NPAD + s * ROWS_T, ROWS_T)])


def _degrees(dst):
    k = pl.kernel(
        _deg_kernel,
        out_type=jax.ShapeDtypeStruct((NC * NPAD,), jnp.float32),
        mesh=plsc.VectorSubcoreMesh(core_axis_name="c",
                                    subcore_axis_name="s"),
        scratch_types=(
            [pltpu.VMEM((NCHUNK, CHUNK), jnp.int32),
             pltpu.VMEM((CHUNK,), jnp.float32),
             pltpu.VMEM((ROWS_T,), jnp.float32)]
            + [pltpu.SemaphoreType.DMA] * 4
            + [pltpu.VMEM_SHARED((NPAD,), jnp.float32)]
        ),
    )
    return k(dst.reshape(NW, NCHUNK, CHUNK))


def _xw_kernel(x_ref, w_ref, d0_ref, d1_ref, y_ref, dinv_ref):
    deg = d0_ref[...] + d1_ref[...] + 1.0
    dinv = lax.rsqrt(deg)
    xw = jnp.dot(x_ref[...], w_ref[...],
                 preferred_element_type=jnp.float32)
    y_ref[...] = xw * dinv
    dinv_ref[...] = dinv


def _compute_y(x, w, d0, d1):
    blk = 2048
    grid = (N + blk - 1) // blk
    return pl.pallas_call(
        _xw_kernel,
        grid=(grid,),
        out_shape=(jax.ShapeDtypeStruct((N, F), jnp.float32),
                   jax.ShapeDtypeStruct((N, 1), jnp.float32)),
        in_specs=[
            pl.BlockSpec((blk, F), lambda i: (i, 0)),
            pl.BlockSpec((F, F), lambda i: (0, 0)),
            pl.BlockSpec((blk, 1), lambda i: (i, 0)),
            pl.BlockSpec((blk, 1), lambda i: (i, 0)),
        ],
        out_specs=(pl.BlockSpec((blk, F), lambda i: (i, 0)),
                   pl.BlockSpec((blk, 1), lambda i: (i, 0))),
    )(x, w, d0, d1)


def _agg_kernel(src_hbm, dst_hbm, y_hbm, part_out,
                r0, r1, r2, r3, r4, r5,
                g0, g1, g2, g3, g4, g5, s0, s1, s2, s3, s4, s5,
                si0, si1, si2, si3, si4, si5, si6, si7, si8, si9, si10,
                si11,
                di0, di1, di2, di3, di4, di5, di6, di7, di8, di9, di10,
                di11,
                i0, i1, i2, i3, i4, i5, i6, i7, i8, i9, i10, i11,
                zr0, zr1, o0, o1, acc_sh):
    rows = [r0, r1, r2, r3, r4, r5]
    gsem = [g0, g1, g2, g3, g4, g5]
    ssem = [s0, s1, s2, s3, s4, s5]
    sidx = [si0, si1, si2, si3, si4, si5, si6, si7, si8, si9, si10, si11]
    didx = [di0, di1, di2, di3, di4, di5, di6, di7, di8, di9, di10, di11]
    isem = [i0, i1, i2, i3, i4, i5, i6, i7, i8, i9, i10, i11]
    zr = [zr0, zr1]
    osem = [o0, o1]
    c = lax.axis_index("c")
    s = lax.axis_index("s")
    wid = c * NS + s
    ebase = wid * EDGES_W

    def idx_fire(k, slot):
        pltpu.async_copy(src_hbm.at[pl.ds(ebase + k * C4, C4)],
                         sidx[slot], isem[slot])
        pltpu.async_copy(dst_hbm.at[pl.ds(ebase + k * C4, C4)],
                         didx[slot], isem[slot])

    def idx_wait(k, slot):
        pltpu.make_async_copy(src_hbm.at[pl.ds(ebase, C4)],
                              sidx[slot], isem[slot]).wait()
        pltpu.make_async_copy(dst_hbm.at[pl.ds(ebase, C4)],
                              didx[slot], isem[slot]).wait()

    for k in range(2 * NB4):
        idx_fire(k, k)
    for b in range(NB4):
        idx_wait(b, b)
        pltpu.async_copy(y_hbm.at[sidx[b]], rows[b], gsem[b])

    zv = jnp.zeros((16,), jnp.float32)
    for rr in range(ZROWS):
        for cc in range(F // 16):
            zr0[rr, pl.ds(cc * 16, 16)] = zv
    for q in range(ROWS_T // ZROWS):
        pltpu.async_copy(
            zr0, acc_sh.at[pl.ds(s * ROWS_T + q * ZROWS, ZROWS)], o0)
    for q in range(ROWS_T // ZROWS):
        pltpu.make_async_copy(
            zr0, acc_sh.at[pl.ds(s * ROWS_T, ZROWS)], o0).wait()
    plsc.subcore_barrier()

    def group(g, _):
        for bb in range(MI4):
            b = bb % M4
            sf = (bb + NB4) % M4
            iu = (bb + NB4) % MI4
            ifi = (bb + 2 * NB4) % MI4
            j = g * MI4 + bb

            @pl.when(j < NCH4)
            def _():
                pltpu.make_async_copy(y_hbm.at[sidx[0]], rows[b],
                                      gsem[b]).wait()
                pltpu.async_copy(rows[b], acc_sh.at[didx[bb % MI4]],
                                 ssem[b], add=True)

            @pl.when(jnp.logical_and(j >= M4 - NB4, j + NB4 < NCH4))
            def _():
                pltpu.make_async_copy(rows[sf], acc_sh.at[didx[0]],
                                      ssem[sf]).wait()

            @pl.when(j + NB4 < NCH4)
            def _():
                idx_wait(j + NB4, iu)
                pltpu.async_copy(y_hbm.at[sidx[iu]], rows[sf], gsem[sf])

            @pl.when(j + 2 * NB4 < NCH4)
            def _():
                idx_fire(j + 2 * NB4, ifi)
        return 0

    lax.fori_loop(0, (NCH4 + MI4 - 1) // MI4, group, 0)
    for b in range(M4):
        pltpu.make_async_copy(rows[b], acc_sh.at[didx[0]],
                              ssem[b]).wait()
    plsc.subcore_barrier()

    NQ = ROWS_T // ZROWS
    for q in range(NQ):
        b = q % 2
        if q >= 2:
            pltpu.make_async_copy(
                zr[b], part_out.at[pl.ds(c * NPAD, ZROWS)], osem[b]).wait()
        off = s * ROWS_T + q * ZROWS
        pltpu.sync_copy(acc_sh.at[pl.ds(off, ZROWS)], zr[b])
        pltpu.async_copy(zr[b], part_out.at[pl.ds(c * NPAD + off, ZROWS)],
                         osem[b])
    for b in range(2):
        pltpu.make_async_copy(
            zr[b], part_out.at[pl.ds(c * NPAD, ZROWS)], osem[b]).wait()


def _aggregate(src, dst, y):
    k = pl.kernel(
        _agg_kernel,
        out_type=jax.ShapeDtypeStruct((NC * NPAD, F), jnp.float32),
        mesh=plsc.VectorSubcoreMesh(core_axis_name="c",
                                    subcore_axis_name="s"),
        scratch_types=(
            [pltpu.VMEM((C4, F), jnp.float32)] * 6
            + [pltpu.SemaphoreType.DMA] * 12
            + [pltpu.VMEM((C4,), jnp.int32)] * 24
            + [pltpu.SemaphoreType.DMA] * 12
            + [pltpu.VMEM((ZROWS, F), jnp.float32)] * 2
            + [pltpu.SemaphoreType.DMA] * 2
            + [pltpu.VMEM_SHARED((NPAD, F), jnp.float32)]
        ),
    )
    return k(src, dst, y)


def _head_kernel(p_ref, y_ref, dinv_ref, lw_ref, lb_ref, out_ref):
    t = p_ref[0] + p_ref[1] + y_ref[...]
    h = jnp.maximum(t, 0.0) * dinv_ref[...]
    out_ref[...] = lax.dot_general(
        h, lw_ref[...], (((1,), (1,)), ((), ())),
        preferred_element_type=jnp.float32) + lb_ref[...]


def _head(parts, y, dinv, lin_w, lin_b):
    blk = 2048
    grid = NPAD // blk
    return pl.pallas_call(
        _head_kernel,
        grid=(grid,),
        out_shape=jax.ShapeDtypeStruct((N, L), jnp.float32),
        in_specs=[
            pl.BlockSpec((NC, blk, F), lambda i: (0, i, 0)),
            pl.BlockSpec((blk, F), lambda i: (i, 0)),
            pl.BlockSpec((blk, 1), lambda i: (i, 0)),
            pl.BlockSpec((L, F), lambda i: (0, 0)),
            pl.BlockSpec((1, L), lambda i: (0, 0)),
        ],
        out_specs=pl.BlockSpec((blk, L), lambda i: (i, 0)),
    )(parts, y, dinv, lin_w, lin_b.reshape(1, L))


def kernel(x, edge_index, pool_p, W0, gru_w_ih, gru_w_hh, gru_b_ih,
           gru_b_hh, lin_w, lin_b):
    src = edge_index[0]
    dst = edge_index[1]
    s_col = _score(x, pool_p)
    score2d = jnp.pad(s_col[:, 0], (0, NPAD - N),
                      constant_values=NEG).reshape(SROWS, 128)
    w = _evolve_w(score2d, x, W0, gru_w_ih, gru_w_hh, gru_b_ih, gru_b_hh)
    deg2 = _degrees(dst)
    d0 = deg2[:N].reshape(N, 1)
    d1 = deg2[NPAD:NPAD + N].reshape(N, 1)
    y, dinv = _compute_y(x, w, d0, d1)
    parts = _aggregate(src, dst, y).reshape(NC, NPAD, F)
    return _head(parts, y, dinv, lin_w, lin_b)


# speedup vs baseline: 31.6685x; 1.0032x over previous
import jax
import jax.numpy as jnp
from jax import lax
from jax.experimental import pallas as pl
from jax.experimental.pallas import tpu as pltpu
from jax.experimental.pallas import tpu_sc as plsc

N = 10000
F = 128
L = 64
E = 320000

NC = 2
NS = 16
NW = NC * NS

NPAD = 10240
ROWS_T = NPAD // NS
EDGES_W = E // NW
CHUNK = 80
NCHUNK = EDGES_W // CHUNK
C4 = 40
NCH4 = EDGES_W // C4
M4 = 6
NB4 = 3
MI4 = 12
SROWS = NPAD // 128
ZROWS = 16

NEG = -3.0e38


def _score_kernel(x_ref, p_ref, s_ref):
    p = p_ref[...]
    pnorm = jnp.sqrt(jnp.sum(p * p))
    xb = x_ref[...].astype(jnp.bfloat16).astype(jnp.float32)
    pb = p.astype(jnp.bfloat16).astype(jnp.float32)
    s = lax.dot_general(xb, pb, (((1,), (1,)), ((), ())),
                        preferred_element_type=jnp.float32)
    s_ref[...] = s / pnorm


def _score(x, pool_p):
    return pl.pallas_call(
        _score_kernel,
        out_shape=jax.ShapeDtypeStruct((N, 1), jnp.float32),
    )(x, pool_p.reshape(1, F))


def _evolve_kernel(s_ref, x_ref, w0_ref, wih_ref, whh_ref, bih_ref,
                   bhh_ref, w_out_ref, perm_ref, topv_ref, xt_ref):
    idx2 = (lax.broadcasted_iota(jnp.int32, (SROWS, 128), 0) * 128
            + lax.broadcasted_iota(jnp.int32, (SROWS, 128), 1))

    def topk_body(i, sc):
        m = jnp.max(sc)
        am = jnp.min(jnp.where(sc == m, idx2, jnp.int32(2 ** 30)))
        perm_ref[i] = am
        topv_ref[i] = m
        return jnp.where(idx2 == am, NEG, sc)

    lax.fori_loop(0, F, topk_body, s_ref[...])

    def gather_body(i, _):
        pi = perm_ref[i]
        tv = topv_ref[i]
        row = x_ref[pl.ds(pi, 1), :]
        xt_ref[pl.ds(i, 1), :] = row * jnp.tanh(
            jnp.broadcast_to(tv, (1, F)))
        return 0

    lax.fori_loop(0, F, gather_body, 0)

    xt = xt_ref[...]
    w0 = w0_ref[...]
    gi = lax.dot_general(xt, wih_ref[...], (((1,), (1,)), ((), ())),
                         preferred_element_type=jnp.float32) + bih_ref[...]
    gh = lax.dot_general(w0, whh_ref[...], (((1,), (1,)), ((), ())),
                         preferred_element_type=jnp.float32) + bhh_ref[...]
    i_r, i_z, i_n = gi[:, :F], gi[:, F:2 * F], gi[:, 2 * F:]
    h_r, h_z, h_n = gh[:, :F], gh[:, F:2 * F], gh[:, 2 * F:]
    r = jax.nn.sigmoid(i_r + h_r)
    z = jax.nn.sigmoid(i_z + h_z)
    n = jnp.tanh(i_n + r * h_n)
    w_out_ref[...] = (1.0 - z) * n + z * w0


def _evolve_w(score2d, x, w0, wih, whh, bih, bhh):
    return pl.pallas_call(
        _evolve_kernel,
        out_shape=jax.ShapeDtypeStruct((F, F), jnp.float32),
        scratch_shapes=[
            pltpu.SMEM((F,), jnp.int32),
            pltpu.SMEM((F,), jnp.float32),
            pltpu.VMEM((F, F), jnp.float32),
        ],
    )(score2d, x, w0, wih, whh, bih.reshape(1, 3 * F),
      bhh.reshape(1, 3 * F))


def _deg_kernel(dst2_hbm, deg_out, didx_all, ones_v, zero_v,
                s0, s1, s2, s3, deg_sh):
    ssem = [s0, s1, s2, s3]
    c = lax.axis_index("c")
    s = lax.axis_index("s")
    wid = c * NS + s
    pltpu.sync_copy(dst2_hbm.at[wid], didx_all)
    zv = jnp.zeros((16,), jnp.float32)
    for q in range(ROWS_T // 16):
        zero_v[pl.ds(q * 16, 16)] = zv
    ov = jnp.ones((16,), jnp.float32)
    for q in range(CHUNK // 16):
        ones_v[pl.ds(q * 16, 16)] = ov
    pltpu.sync_copy(zero_v, deg_sh.at[pl.ds(s * ROWS_T, ROWS_T)])
    plsc.subcore_barrier()

    def visit(g, _):
        for b in range(4):
            j = g * 4 + b

            @pl.when(jnp.logical_and(j >= 4, j < NCHUNK))
            def _():
                pltpu.make_async_copy(ones_v, deg_sh.at[didx_all.at[0]],
                                      ssem[b]).wait()

            @pl.when(j < NCHUNK)
            def _():
                pltpu.async_copy(ones_v, deg_sh.at[didx_all.at[j]],
                                 ssem[b], add=True)
        return 0

    lax.fori_loop(0, (NCHUNK + 3) // 4, visit, 0)
    for b in range(4):
        pltpu.make_async_copy(ones_v, deg_sh.at[didx_all.at[0]],
                              ssem[b]).wait()
    plsc.subcore_barrier()
    pltpu.sync_copy(deg_sh.at[pl.ds(s * ROWS_T, ROWS_T)], zero_v)
    pltpu.sync_copy(zero_v,
                    deg_out.at[pl.ds(c * NPAD + s * ROWS_T, ROWS_T)])


def _degrees(dst):
    k = pl.kernel(
        _deg_kernel,
        out_type=jax.ShapeDtypeStruct((NC * NPAD,), jnp.float32),
        mesh=plsc.VectorSubcoreMesh(core_axis_name="c",
                                    subcore_axis_name="s"),
        scratch_types=(
            [pltpu.VMEM((NCHUNK, CHUNK), jnp.int32),
             pltpu.VMEM((CHUNK,), jnp.float32),
             pltpu.VMEM((ROWS_T,), jnp.float32)]
            + [pltpu.SemaphoreType.DMA] * 4
            + [pltpu.VMEM_SHARED((NPAD,), jnp.float32)]
        ),
    )
    return k(dst.reshape(NW, NCHUNK, CHUNK))


def _xw_kernel(x_ref, w_ref, d0_ref, d1_ref, y_ref, dinv_ref):
    deg = d0_ref[...] + d1_ref[...] + 1.0
    dinv = lax.rsqrt(deg)
    xw = jnp.dot(x_ref[...], w_ref[...],
                 preferred_element_type=jnp.float32)
    y_ref[...] = xw * dinv
    dinv_ref[...] = dinv


def _compute_y(x, w, d0, d1):
    blk = 2048
    grid = (N + blk - 1) // blk
    return pl.pallas_call(
        _xw_kernel,
        grid=(grid,),
        out_shape=(jax.ShapeDtypeStruct((N, F), jnp.float32),
                   jax.ShapeDtypeStruct((N, 1), jnp.float32)),
        in_specs=[
            pl.BlockSpec((blk, F), lambda i: (i, 0)),
            pl.BlockSpec((F, F), lambda i: (0, 0)),
            pl.BlockSpec((blk, 1), lambda i: (i, 0)),
            pl.BlockSpec((blk, 1), lambda i: (i, 0)),
        ],
        out_specs=(pl.BlockSpec((blk, F), lambda i: (i, 0)),
                   pl.BlockSpec((blk, 1), lambda i: (i, 0))),
    )(x, w, d0, d1)


def _agg_kernel(src_hbm, dst_hbm, y_hbm, part_out,
                r0, r1, r2, r3, r4, r5,
                g0, g1, g2, g3, g4, g5, s0, s1, s2, s3, s4, s5,
                si0, si1, si2, si3, si4, si5, si6, si7, si8, si9, si10,
                si11,
                di0, di1, di2, di3, di4, di5, di6, di7, di8, di9, di10,
                di11,
                i0, i1, i2, i3, i4, i5, i6, i7, i8, i9, i10, i11,
                zr0, zr1, o0, o1, acc_sh):
    rows = [r0, r1, r2, r3, r4, r5]
    gsem = [g0, g1, g2, g3, g4, g5]
    ssem = [s0, s1, s2, s3, s4, s5]
    sidx = [si0, si1, si2, si3, si4, si5, si6, si7, si8, si9, si10, si11]
    didx = [di0, di1, di2, di3, di4, di5, di6, di7, di8, di9, di10, di11]
    isem = [i0, i1, i2, i3, i4, i5, i6, i7, i8, i9, i10, i11]
    zr = [zr0, zr1]
    osem = [o0, o1]
    c = lax.axis_index("c")
    s = lax.axis_index("s")
    wid = c * NS + s
    ebase = wid * EDGES_W

    def idx_fire(k, slot):
        pltpu.async_copy(src_hbm.at[pl.ds(ebase + k * C4, C4)],
                         sidx[slot], isem[slot])
        pltpu.async_copy(dst_hbm.at[pl.ds(ebase + k * C4, C4)],
                         didx[slot], isem[slot])

    def idx_wait(k, slot):
        pltpu.make_async_copy(src_hbm.at[pl.ds(ebase, C4)],
                              sidx[slot], isem[slot]).wait()
        pltpu.make_async_copy(dst_hbm.at[pl.ds(ebase, C4)],
                              didx[slot], isem[slot]).wait()

    for k in range(2 * NB4):
        idx_fire(k, k)
    for b in range(NB4):
        idx_wait(b, b)
        pltpu.async_copy(y_hbm.at[sidx[b]], rows[b], gsem[b])

    zv = jnp.zeros((16,), jnp.float32)
    for rr in range(ZROWS):
        for cc in range(F // 16):
            zr0[rr, pl.ds(cc * 16, 16)] = zv
    for q in range(ROWS_T // ZROWS):
        pltpu.async_copy(
            zr0, acc_sh.at[pl.ds(s * ROWS_T + q * ZROWS, ZROWS)], o0)
    for q in range(ROWS_T // ZROWS):
        pltpu.make_async_copy(
            zr0, acc_sh.at[pl.ds(s * ROWS_T, ZROWS)], o0).wait()
    plsc.subcore_barrier()

    def group(g, _):
        for bb in range(MI4):
            b = bb % M4
            sf = (bb + NB4) % M4
            iu = (bb + NB4) % MI4
            ifi = (bb + 2 * NB4) % MI4
            j = g * MI4 + bb

            @pl.when(j < NCH4)
            def _():
                pltpu.make_async_copy(y_hbm.at[sidx[0]], rows[b],
                                      gsem[b]).wait()
                pltpu.async_copy(rows[b], acc_sh.at[didx[bb % MI4]],
                                 ssem[b], add=True)

            @pl.when(jnp.logical_and(j >= M4 - NB4, j + NB4 < NCH4))
            def _():
                pltpu.make_async_copy(rows[sf], acc_sh.at[didx[0]],
                                      ssem[sf]).wait()

            @pl.when(j + NB4 < NCH4)
            def _():
                idx_wait(j + NB4, iu)
                pltpu.async_copy(y_hbm.at[sidx[iu]], rows[sf], gsem[sf])

            @pl.when(j + 2 * NB4 < NCH4)
            def _():
                idx_fire(j + 2 * NB4, ifi)
        return 0

    lax.fori_loop(0, (NCH4 + MI4 - 1) // MI4, group, 0)
    for b in range(M4):
        pltpu.make_async_copy(rows[b], acc_sh.at[didx[0]],
                              ssem[b]).wait()
    plsc.subcore_barrier()

    NQ = ROWS_T // ZROWS
    for q in range(NQ):
        b = q % 2
        if q >= 2:
            pltpu.make_async_copy(
                zr[b], part_out.at[pl.ds(c * NPAD, ZROWS)], osem[b]).wait()
        off = s * ROWS_T + q * ZROWS
        pltpu.sync_copy(acc_sh.at[pl.ds(off, ZROWS)], zr[b])
        pltpu.async_copy(zr[b], part_out.at[pl.ds(c * NPAD + off, ZROWS)],
                         osem[b])
    for b in range(2):
        pltpu.make_async_copy(
            zr[b], part_out.at[pl.ds(c * NPAD, ZROWS)], osem[b]).wait()


def _aggregate(src, dst, y):
    k = pl.kernel(
        _agg_kernel,
        out_type=jax.ShapeDtypeStruct((NC * NPAD, F), jnp.float32),
        mesh=plsc.VectorSubcoreMesh(core_axis_name="c",
                                    subcore_axis_name="s"),
        scratch_types=(
            [pltpu.VMEM((C4, F), jnp.float32)] * 6
            + [pltpu.SemaphoreType.DMA] * 12
            + [pltpu.VMEM((C4,), jnp.int32)] * 24
            + [pltpu.SemaphoreType.DMA] * 12
            + [pltpu.VMEM((ZROWS, F), jnp.float32)] * 2
            + [pltpu.SemaphoreType.DMA] * 2
            + [pltpu.VMEM_SHARED((NPAD, F), jnp.float32)]
        ),
    )
    return k(src, dst, y)


def _head_kernel(p_ref, y_ref, dinv_ref, lw_ref, lb_ref, out_ref):
    t = p_ref[0] + p_ref[1] + y_ref[...]
    h = jnp.maximum(t, 0.0) * dinv_ref[...]
    out_ref[...] = lax.dot_general(
        h, lw_ref[...], (((1,), (1,)), ((), ())),
        preferred_element_type=jnp.float32) + lb_ref[...]


def _head(parts, y, dinv, lin_w, lin_b):
    blk = 2048
    grid = NPAD // blk
    return pl.pallas_call(
        _head_kernel,
        grid=(grid,),
        out_shape=jax.ShapeDtypeStruct((N, L), jnp.float32),
        in_specs=[
            pl.BlockSpec((NC, blk, F), lambda i: (0, i, 0)),
            pl.BlockSpec((blk, F), lambda i: (i, 0)),
            pl.BlockSpec((blk, 1), lambda i: (i, 0)),
            pl.BlockSpec((L, F), lambda i: (0, 0)),
            pl.BlockSpec((1, L), lambda i: (0, 0)),
        ],
        out_specs=pl.BlockSpec((blk, L), lambda i: (i, 0)),
    )(parts, y, dinv, lin_w, lin_b.reshape(1, L))


def kernel(x, edge_index, pool_p, W0, gru_w_ih, gru_w_hh, gru_b_ih,
           gru_b_hh, lin_w, lin_b):
    src = edge_index[0]
    dst = edge_index[1]
    s_col = _score(x, pool_p)
    score2d = jnp.pad(s_col[:, 0], (0, NPAD - N),
                      constant_values=NEG).reshape(SROWS, 128)
    w = _evolve_w(score2d, x, W0, gru_w_ih, gru_w_hh, gru_b_ih, gru_b_hh)
    deg2 = _degrees(dst)
    d0 = deg2[:N].reshape(N, 1)
    d1 = deg2[NPAD:NPAD + N].reshape(N, 1)
    y, dinv = _compute_y(x, w, d0, d1)
    parts = _aggregate(src, dst, y).reshape(NC, NPAD, F)
    return _head(parts, y, dinv, lin_w, lin_b)


# speedup vs baseline: 34.5543x; 1.0911x over previous
import jax
import jax.numpy as jnp
from jax import lax
from jax.experimental import pallas as pl
from jax.experimental.pallas import tpu as pltpu
from jax.experimental.pallas import tpu_sc as plsc

N = 10000
F = 128
L = 64
E = 320000

NC = 2
NS = 16
NW = NC * NS

NPAD = 10240
ROWS_T = NPAD // NS
EDGES_W = E // NW
CHUNK = 80
NCHUNK = EDGES_W // CHUNK
C4 = 40
NCH4 = EDGES_W // C4
M4 = 6
NB4 = 4
MI4 = 12
SROWS = NPAD // 128
ZROWS = 16

NEG = -3.0e38


def _score_kernel(x_ref, p_ref, s_ref):
    p = p_ref[...]
    pnorm = jnp.sqrt(jnp.sum(p * p))
    xb = x_ref[...].astype(jnp.bfloat16).astype(jnp.float32)
    pb = p.astype(jnp.bfloat16).astype(jnp.float32)
    s = lax.dot_general(xb, pb, (((1,), (1,)), ((), ())),
                        preferred_element_type=jnp.float32)
    s_ref[...] = s / pnorm


def _score(x, pool_p):
    return pl.pallas_call(
        _score_kernel,
        out_shape=jax.ShapeDtypeStruct((N, 1), jnp.float32),
    )(x, pool_p.reshape(1, F))


def _evolve_kernel(s_ref, x_ref, w0_ref, wih_ref, whh_ref, bih_ref,
                   bhh_ref, w_out_ref, perm_ref, topv_ref, xt_ref):
    idx2 = (lax.broadcasted_iota(jnp.int32, (SROWS, 128), 0) * 128
            + lax.broadcasted_iota(jnp.int32, (SROWS, 128), 1))

    def topk_body(i, sc):
        m = jnp.max(sc)
        am = jnp.min(jnp.where(sc == m, idx2, jnp.int32(2 ** 30)))
        perm_ref[i] = am
        topv_ref[i] = m
        return jnp.where(idx2 == am, NEG, sc)

    lax.fori_loop(0, F, topk_body, s_ref[...])

    def gather_body(i, _):
        pi = perm_ref[i]
        tv = topv_ref[i]
        row = x_ref[pl.ds(pi, 1), :]
        xt_ref[pl.ds(i, 1), :] = row * jnp.tanh(
            jnp.broadcast_to(tv, (1, F)))
        return 0

    lax.fori_loop(0, F, gather_body, 0)

    xt = xt_ref[...]
    w0 = w0_ref[...]
    gi = lax.dot_general(xt, wih_ref[...], (((1,), (1,)), ((), ())),
                         preferred_element_type=jnp.float32) + bih_ref[...]
    gh = lax.dot_general(w0, whh_ref[...], (((1,), (1,)), ((), ())),
                         preferred_element_type=jnp.float32) + bhh_ref[...]
    i_r, i_z, i_n = gi[:, :F], gi[:, F:2 * F], gi[:, 2 * F:]
    h_r, h_z, h_n = gh[:, :F], gh[:, F:2 * F], gh[:, 2 * F:]
    r = jax.nn.sigmoid(i_r + h_r)
    z = jax.nn.sigmoid(i_z + h_z)
    n = jnp.tanh(i_n + r * h_n)
    w_out_ref[...] = (1.0 - z) * n + z * w0


def _evolve_w(score2d, x, w0, wih, whh, bih, bhh):
    return pl.pallas_call(
        _evolve_kernel,
        out_shape=jax.ShapeDtypeStruct((F, F), jnp.float32),
        scratch_shapes=[
            pltpu.SMEM((F,), jnp.int32),
            pltpu.SMEM((F,), jnp.float32),
            pltpu.VMEM((F, F), jnp.float32),
        ],
    )(score2d, x, w0, wih, whh, bih.reshape(1, 3 * F),
      bhh.reshape(1, 3 * F))


def _deg_kernel(dst2_hbm, deg_out, didx_all, ones_v, zero_v,
                s0, s1, s2, s3, deg_sh):
    ssem = [s0, s1, s2, s3]
    c = lax.axis_index("c")
    s = lax.axis_index("s")
    wid = c * NS + s
    pltpu.sync_copy(dst2_hbm.at[wid], didx_all)
    zv = jnp.zeros((16,), jnp.float32)
    for q in range(ROWS_T // 16):
        zero_v[pl.ds(q * 16, 16)] = zv
    ov = jnp.ones((16,), jnp.float32)
    for q in range(CHUNK // 16):
        ones_v[pl.ds(q * 16, 16)] = ov
    pltpu.sync_copy(zero_v, deg_sh.at[pl.ds(s * ROWS_T, ROWS_T)])
    plsc.subcore_barrier()

    def visit(g, _):
        for b in range(4):
            j = g * 4 + b

            @pl.when(jnp.logical_and(j >= 4, j < NCHUNK))
            def _():
                pltpu.make_async_copy(ones_v, deg_sh.at[didx_all.at[0]],
                                      ssem[b]).wait()

            @pl.when(j < NCHUNK)
            def _():
                pltpu.async_copy(ones_v, deg_sh.at[didx_all.at[j]],
                                 ssem[b], add=True)
        return 0

    lax.fori_loop(0, (NCHUNK + 3) // 4, visit, 0)
    for b in range(4):
        pltpu.make_async_copy(ones_v, deg_sh.at[didx_all.at[0]],
                              ssem[b]).wait()
    plsc.subcore_barrier()
    pltpu.sync_copy(deg_sh.at[pl.ds(s * ROWS_T, ROWS_T)], zero_v)
    pltpu.sync_copy(zero_v,
                    deg_out.at[pl.ds(c * NPAD + s * ROWS_T, ROWS_T)])


def _degrees(dst):
    k = pl.kernel(
        _deg_kernel,
        out_type=jax.ShapeDtypeStruct((NC * NPAD,), jnp.float32),
        mesh=plsc.VectorSubcoreMesh(core_axis_name="c",
                                    subcore_axis_name="s"),
        scratch_types=(
            [pltpu.VMEM((NCHUNK, CHUNK), jnp.int32),
             pltpu.VMEM((CHUNK,), jnp.float32),
             pltpu.VMEM((ROWS_T,), jnp.float32)]
            + [pltpu.SemaphoreType.DMA] * 4
            + [pltpu.VMEM_SHARED((NPAD,), jnp.float32)]
        ),
    )
    return k(dst.reshape(NW, NCHUNK, CHUNK))


def _xw_kernel(x_ref, w_ref, d0_ref, d1_ref, y_ref, dinv_ref):
    deg = d0_ref[...] + d1_ref[...] + 1.0
    dinv = lax.rsqrt(deg)
    xw = jnp.dot(x_ref[...], w_ref[...],
                 preferred_element_type=jnp.float32)
    y_ref[...] = xw * dinv
    dinv_ref[...] = dinv


def _compute_y(x, w, d0, d1):
    blk = 2048
    grid = (N + blk - 1) // blk
    return pl.pallas_call(
        _xw_kernel,
        grid=(grid,),
        out_shape=(jax.ShapeDtypeStruct((N, F), jnp.float32),
                   jax.ShapeDtypeStruct((N, 1), jnp.float32)),
        in_specs=[
            pl.BlockSpec((blk, F), lambda i: (i, 0)),
            pl.BlockSpec((F, F), lambda i: (0, 0)),
            pl.BlockSpec((blk, 1), lambda i: (i, 0)),
            pl.BlockSpec((blk, 1), lambda i: (i, 0)),
        ],
        out_specs=(pl.BlockSpec((blk, F), lambda i: (i, 0)),
                   pl.BlockSpec((blk, 1), lambda i: (i, 0))),
    )(x, w, d0, d1)


def _agg_kernel(src_hbm, dst_hbm, y_hbm, part_out,
                r0, r1, r2, r3, r4, r5,
                g0, g1, g2, g3, g4, g5, s0, s1, s2, s3, s4, s5,
                si0, si1, si2, si3, si4, si5, si6, si7, si8, si9, si10,
                si11,
                di0, di1, di2, di3, di4, di5, di6, di7, di8, di9, di10,
                di11,
                i0, i1, i2, i3, i4, i5, i6, i7, i8, i9, i10, i11,
                zr0, zr1, o0, o1, acc_sh):
    rows = [r0, r1, r2, r3, r4, r5]
    gsem = [g0, g1, g2, g3, g4, g5]
    ssem = [s0, s1, s2, s3, s4, s5]
    sidx = [si0, si1, si2, si3, si4, si5, si6, si7, si8, si9, si10, si11]
    didx = [di0, di1, di2, di3, di4, di5, di6, di7, di8, di9, di10, di11]
    isem = [i0, i1, i2, i3, i4, i5, i6, i7, i8, i9, i10, i11]
    zr = [zr0, zr1]
    osem = [o0, o1]
    c = lax.axis_index("c")
    s = lax.axis_index("s")
    wid = c * NS + s
    ebase = wid * EDGES_W

    def idx_fire(k, slot):
        pltpu.async_copy(src_hbm.at[pl.ds(ebase + k * C4, C4)],
                         sidx[slot], isem[slot])
        pltpu.async_copy(dst_hbm.at[pl.ds(ebase + k * C4, C4)],
                         didx[slot], isem[slot])

    def idx_wait(k, slot):
        pltpu.make_async_copy(src_hbm.at[pl.ds(ebase, C4)],
                              sidx[slot], isem[slot]).wait()
        pltpu.make_async_copy(dst_hbm.at[pl.ds(ebase, C4)],
                              didx[slot], isem[slot]).wait()

    for k in range(2 * NB4):
        idx_fire(k, k)
    for b in range(NB4):
        idx_wait(b, b)
        pltpu.async_copy(y_hbm.at[sidx[b]], rows[b], gsem[b])

    zv = jnp.zeros((16,), jnp.float32)
    for rr in range(ZROWS):
        for cc in range(F // 16):
            zr0[rr, pl.ds(cc * 16, 16)] = zv
    for q in range(ROWS_T // ZROWS):
        pltpu.async_copy(
            zr0, acc_sh.at[pl.ds(s * ROWS_T + q * ZROWS, ZROWS)], o0)
    for q in range(ROWS_T // ZROWS):
        pltpu.make_async_copy(
            zr0, acc_sh.at[pl.ds(s * ROWS_T, ZROWS)], o0).wait()
    plsc.subcore_barrier()

    def group(g, _):
        for bb in range(MI4):
            b = bb % M4
            sf = (bb + NB4) % M4
            iu = (bb + NB4) % MI4
            ifi = (bb + 2 * NB4) % MI4
            j = g * MI4 + bb

            @pl.when(j < NCH4)
            def _():
                pltpu.make_async_copy(y_hbm.at[sidx[0]], rows[b],
                                      gsem[b]).wait()
                pltpu.async_copy(rows[b], acc_sh.at[didx[bb % MI4]],
                                 ssem[b], add=True)

            @pl.when(jnp.logical_and(j >= M4 - NB4, j + NB4 < NCH4))
            def _():
                pltpu.make_async_copy(rows[sf], acc_sh.at[didx[0]],
                                      ssem[sf]).wait()

            @pl.when(j + NB4 < NCH4)
            def _():
                idx_wait(j + NB4, iu)
                pltpu.async_copy(y_hbm.at[sidx[iu]], rows[sf], gsem[sf])

            @pl.when(j + 2 * NB4 < NCH4)
            def _():
                idx_fire(j + 2 * NB4, ifi)
        return 0

    lax.fori_loop(0, (NCH4 + MI4 - 1) // MI4, group, 0)
    for b in range(M4):
        pltpu.make_async_copy(rows[b], acc_sh.at[didx[0]],
                              ssem[b]).wait()
    plsc.subcore_barrier()

    NQ = ROWS_T // ZROWS
    for q in range(NQ):
        b = q % 2
        if q >= 2:
            pltpu.make_async_copy(
                zr[b], part_out.at[pl.ds(c * NPAD, ZROWS)], osem[b]).wait()
        off = s * ROWS_T + q * ZROWS
        pltpu.sync_copy(acc_sh.at[pl.ds(off, ZROWS)], zr[b])
        pltpu.async_copy(zr[b], part_out.at[pl.ds(c * NPAD + off, ZROWS)],
                         osem[b])
    for b in range(2):
        pltpu.make_async_copy(
            zr[b], part_out.at[pl.ds(c * NPAD, ZROWS)], osem[b]).wait()


def _aggregate(src, dst, y):
    k = pl.kernel(
        _agg_kernel,
        out_type=jax.ShapeDtypeStruct((NC * NPAD, F), jnp.float32),
        mesh=plsc.VectorSubcoreMesh(core_axis_name="c",
                                    subcore_axis_name="s"),
        scratch_types=(
            [pltpu.VMEM((C4, F), jnp.float32)] * 6
            + [pltpu.SemaphoreType.DMA] * 12
            + [pltpu.VMEM((C4,), jnp.int32)] * 24
            + [pltpu.SemaphoreType.DMA] * 12
            + [pltpu.VMEM((ZROWS, F), jnp.float32)] * 2
            + [pltpu.SemaphoreType.DMA] * 2
            + [pltpu.VMEM_SHARED((NPAD, F), jnp.float32)]
        ),
    )
    return k(src, dst, y)


def _head_kernel(p_ref, y_ref, dinv_ref, lw_ref, lb_ref, out_ref):
    t = p_ref[0] + p_ref[1] + y_ref[...]
    h = jnp.maximum(t, 0.0) * dinv_ref[...]
    out_ref[...] = lax.dot_general(
        h, lw_ref[...], (((1,), (1,)), ((), ())),
        preferred_element_type=jnp.float32) + lb_ref[...]


def _head(parts, y, dinv, lin_w, lin_b):
    blk = 2048
    grid = NPAD // blk
    return pl.pallas_call(
        _head_kernel,
        grid=(grid,),
        out_shape=jax.ShapeDtypeStruct((N, L), jnp.float32),
        in_specs=[
            pl.BlockSpec((NC, blk, F), lambda i: (0, i, 0)),
            pl.BlockSpec((blk, F), lambda i: (i, 0)),
            pl.BlockSpec((blk, 1), lambda i: (i, 0)),
            pl.BlockSpec((L, F), lambda i: (0, 0)),
            pl.BlockSpec((1, L), lambda i: (0, 0)),
        ],
        out_specs=pl.BlockSpec((blk, L), lambda i: (i, 0)),
    )(parts, y, dinv, lin_w, lin_b.reshape(1, L))


def kernel(x, edge_index, pool_p, W0, gru_w_ih, gru_w_hh, gru_b_ih,
           gru_b_hh, lin_w, lin_b):
    src = edge_index[0]
    dst = edge_index[1]
    s_col = _score(x, pool_p)
    score2d = jnp.pad(s_col[:, 0], (0, NPAD - N),
                      constant_values=NEG).reshape(SROWS, 128)
    w = _evolve_w(score2d, x, W0, gru_w_ih, gru_w_hh, gru_b_ih, gru_b_hh)
    deg2 = _degrees(dst)
    d0 = deg2[:N].reshape(N, 1)
    d1 = deg2[NPAD:NPAD + N].reshape(N, 1)
    y, dinv = _compute_y(x, w, d0, d1)
    parts = _aggregate(src, dst, y).reshape(NC, NPAD, F)
    return _head(parts, y, dinv, lin_w, lin_b)


# speedup vs baseline: 35.4625x; 1.0263x over previous
import jax
import jax.numpy as jnp
from jax import lax
from jax.experimental import pallas as pl
from jax.experimental.pallas import tpu as pltpu
from jax.experimental.pallas import tpu_sc as plsc

N = 10000
F = 128
L = 64
E = 320000

NC = 2
NS = 16
NW = NC * NS

NPAD = 10240
ROWS_T = NPAD // NS
EDGES_W = E // NW
CHUNK = 80
NCHUNK = EDGES_W // CHUNK
C4 = 40
NCH4 = EDGES_W // C4
M4 = 6
NB4 = 5
MI4 = 12
SROWS = NPAD // 128
ZROWS = 16

NEG = -3.0e38


def _score_kernel(x_ref, p_ref, s_ref):
    p = p_ref[...]
    pnorm = jnp.sqrt(jnp.sum(p * p))
    xb = x_ref[...].astype(jnp.bfloat16).astype(jnp.float32)
    pb = p.astype(jnp.bfloat16).astype(jnp.float32)
    s = lax.dot_general(xb, pb, (((1,), (1,)), ((), ())),
                        preferred_element_type=jnp.float32)
    s_ref[...] = s / pnorm


def _score(x, pool_p):
    return pl.pallas_call(
        _score_kernel,
        out_shape=jax.ShapeDtypeStruct((N, 1), jnp.float32),
    )(x, pool_p.reshape(1, F))


def _evolve_kernel(s_ref, x_ref, w0_ref, wih_ref, whh_ref, bih_ref,
                   bhh_ref, d0_ref, d1_ref, y_ref, dinv_ref,
                   perm_ref, topv_ref, xt_ref):
    idx2 = (lax.broadcasted_iota(jnp.int32, (SROWS, 128), 0) * 128
            + lax.broadcasted_iota(jnp.int32, (SROWS, 128), 1))

    def topk_body(i, sc):
        m = jnp.max(sc)
        am = jnp.min(jnp.where(sc == m, idx2, jnp.int32(2 ** 30)))
        perm_ref[i] = am
        topv_ref[i] = m
        return jnp.where(idx2 == am, NEG, sc)

    lax.fori_loop(0, F, topk_body, s_ref[...])

    def gather_body(i, _):
        pi = perm_ref[i]
        tv = topv_ref[i]
        row = x_ref[pl.ds(pi, 1), :]
        xt_ref[pl.ds(i, 1), :] = row * jnp.tanh(
            jnp.broadcast_to(tv, (1, F)))
        return 0

    lax.fori_loop(0, F, gather_body, 0)

    xt = xt_ref[...]
    w0 = w0_ref[...]
    gi = lax.dot_general(xt, wih_ref[...], (((1,), (1,)), ((), ())),
                         preferred_element_type=jnp.float32) + bih_ref[...]
    gh = lax.dot_general(w0, whh_ref[...], (((1,), (1,)), ((), ())),
                         preferred_element_type=jnp.float32) + bhh_ref[...]
    i_r, i_z, i_n = gi[:, :F], gi[:, F:2 * F], gi[:, 2 * F:]
    h_r, h_z, h_n = gh[:, :F], gh[:, F:2 * F], gh[:, 2 * F:]
    r = jax.nn.sigmoid(i_r + h_r)
    z = jax.nn.sigmoid(i_z + h_z)
    n = jnp.tanh(i_n + r * h_n)
    w = (1.0 - z) * n + z * w0

    deg = d0_ref[...] + d1_ref[...] + 1.0
    dinv = lax.rsqrt(deg)
    xw = jnp.dot(x_ref[...], w,
                 preferred_element_type=jnp.float32)
    y_ref[...] = xw * dinv
    dinv_ref[...] = dinv


def _evolve_y(score2d, x, w0, wih, whh, bih, bhh, d0, d1):
    return pl.pallas_call(
        _evolve_kernel,
        out_shape=(jax.ShapeDtypeStruct((N, F), jnp.float32),
                   jax.ShapeDtypeStruct((N, 1), jnp.float32)),
        scratch_shapes=[
            pltpu.SMEM((F,), jnp.int32),
            pltpu.SMEM((F,), jnp.float32),
            pltpu.VMEM((F, F), jnp.float32),
        ],
    )(score2d, x, w0, wih, whh, bih.reshape(1, 3 * F),
      bhh.reshape(1, 3 * F), d0, d1)


def _deg_kernel(dst2_hbm, deg_out, didx_all, ones_v, zero_v,
                s0, s1, s2, s3, deg_sh):
    ssem = [s0, s1, s2, s3]
    c = lax.axis_index("c")
    s = lax.axis_index("s")
    wid = c * NS + s
    pltpu.sync_copy(dst2_hbm.at[wid], didx_all)
    zv = jnp.zeros((16,), jnp.float32)
    for q in range(ROWS_T // 16):
        zero_v[pl.ds(q * 16, 16)] = zv
    ov = jnp.ones((16,), jnp.float32)
    for q in range(CHUNK // 16):
        ones_v[pl.ds(q * 16, 16)] = ov
    pltpu.sync_copy(zero_v, deg_sh.at[pl.ds(s * ROWS_T, ROWS_T)])
    plsc.subcore_barrier()

    def visit(g, _):
        for b in range(4):
            j = g * 4 + b

            @pl.when(jnp.logical_and(j >= 4, j < NCHUNK))
            def _():
                pltpu.make_async_copy(ones_v, deg_sh.at[didx_all.at[0]],
                                      ssem[b]).wait()

            @pl.when(j < NCHUNK)
            def _():
                pltpu.async_copy(ones_v, deg_sh.at[didx_all.at[j]],
                                 ssem[b], add=True)
        return 0

    lax.fori_loop(0, (NCHUNK + 3) // 4, visit, 0)
    for b in range(4):
        pltpu.make_async_copy(ones_v, deg_sh.at[didx_all.at[0]],
                              ssem[b]).wait()
    plsc.subcore_barrier()
    pltpu.sync_copy(deg_sh.at[pl.ds(s * ROWS_T, ROWS_T)], zero_v)
    pltpu.sync_copy(zero_v,
                    deg_out.at[pl.ds(c * NPAD + s * ROWS_T, ROWS_T)])


def _degrees(dst):
    k = pl.kernel(
        _deg_kernel,
        out_type=jax.ShapeDtypeStruct((NC * NPAD,), jnp.float32),
        mesh=plsc.VectorSubcoreMesh(core_axis_name="c",
                                    subcore_axis_name="s"),
        scratch_types=(
            [pltpu.VMEM((NCHUNK, CHUNK), jnp.int32),
             pltpu.VMEM((CHUNK,), jnp.float32),
             pltpu.VMEM((ROWS_T,), jnp.float32)]
            + [pltpu.SemaphoreType.DMA] * 4
            + [pltpu.VMEM_SHARED((NPAD,), jnp.float32)]
        ),
    )
    return k(dst.reshape(NW, NCHUNK, CHUNK))


def _agg_kernel(src_hbm, dst_hbm, y_hbm, part_out,
                r0, r1, r2, r3, r4, r5,
                g0, g1, g2, g3, g4, g5, s0, s1, s2, s3, s4, s5,
                si0, si1, si2, si3, si4, si5, si6, si7, si8, si9, si10,
                si11,
                di0, di1, di2, di3, di4, di5, di6, di7, di8, di9, di10,
                di11,
                i0, i1, i2, i3, i4, i5, i6, i7, i8, i9, i10, i11,
                zr0, zr1, o0, o1, acc_sh):
    rows = [r0, r1, r2, r3, r4, r5]
    gsem = [g0, g1, g2, g3, g4, g5]
    ssem = [s0, s1, s2, s3, s4, s5]
    sidx = [si0, si1, si2, si3, si4, si5, si6, si7, si8, si9, si10, si11]
    didx = [di0, di1, di2, di3, di4, di5, di6, di7, di8, di9, di10, di11]
    isem = [i0, i1, i2, i3, i4, i5, i6, i7, i8, i9, i10, i11]
    zr = [zr0, zr1]
    osem = [o0, o1]
    c = lax.axis_index("c")
    s = lax.axis_index("s")
    wid = c * NS + s
    ebase = wid * EDGES_W

    def idx_fire(k, slot):
        pltpu.async_copy(src_hbm.at[pl.ds(ebase + k * C4, C4)],
                         sidx[slot], isem[slot])
        pltpu.async_copy(dst_hbm.at[pl.ds(ebase + k * C4, C4)],
                         didx[slot], isem[slot])

    def idx_wait(k, slot):
        pltpu.make_async_copy(src_hbm.at[pl.ds(ebase, C4)],
                              sidx[slot], isem[slot]).wait()
        pltpu.make_async_copy(dst_hbm.at[pl.ds(ebase, C4)],
                              didx[slot], isem[slot]).wait()

    for k in range(2 * NB4):
        idx_fire(k, k)
    for b in range(NB4):
        idx_wait(b, b)
        pltpu.async_copy(y_hbm.at[sidx[b]], rows[b], gsem[b])

    zv = jnp.zeros((16,), jnp.float32)
    for rr in range(ZROWS):
        for cc in range(F // 16):
            zr0[rr, pl.ds(cc * 16, 16)] = zv
    for q in range(ROWS_T // ZROWS):
        pltpu.async_copy(
            zr0, acc_sh.at[pl.ds(s * ROWS_T + q * ZROWS, ZROWS)], o0)
    for q in range(ROWS_T // ZROWS):
        pltpu.make_async_copy(
            zr0, acc_sh.at[pl.ds(s * ROWS_T, ZROWS)], o0).wait()
    plsc.subcore_barrier()

    def group(g, _):
        for bb in range(MI4):
            b = bb % M4
            sf = (bb + NB4) % M4
            iu = (bb + NB4) % MI4
            ifi = (bb + 2 * NB4) % MI4
            j = g * MI4 + bb

            @pl.when(j < NCH4)
            def _():
                pltpu.make_async_copy(y_hbm.at[sidx[0]], rows[b],
                                      gsem[b]).wait()
                pltpu.async_copy(rows[b], acc_sh.at[didx[bb % MI4]],
                                 ssem[b], add=True)

            @pl.when(jnp.logical_and(j >= M4 - NB4, j + NB4 < NCH4))
            def _():
                pltpu.make_async_copy(rows[sf], acc_sh.at[didx[0]],
                                      ssem[sf]).wait()

            @pl.when(j + NB4 < NCH4)
            def _():
                idx_wait(j + NB4, iu)
                pltpu.async_copy(y_hbm.at[sidx[iu]], rows[sf], gsem[sf])

            @pl.when(j + 2 * NB4 < NCH4)
            def _():
                idx_fire(j + 2 * NB4, ifi)
        return 0

    lax.fori_loop(0, (NCH4 + MI4 - 1) // MI4, group, 0)
    for b in range(M4):
        pltpu.make_async_copy(rows[b], acc_sh.at[didx[0]],
                              ssem[b]).wait()
    plsc.subcore_barrier()

    NQ = ROWS_T // ZROWS
    for q in range(NQ):
        b = q % 2
        if q >= 2:
            pltpu.make_async_copy(
                zr[b], part_out.at[pl.ds(c * NPAD, ZROWS)], osem[b]).wait()
        off = s * ROWS_T + q * ZROWS
        pltpu.sync_copy(acc_sh.at[pl.ds(off, ZROWS)], zr[b])
        pltpu.async_copy(zr[b], part_out.at[pl.ds(c * NPAD + off, ZROWS)],
                         osem[b])
    for b in range(2):
        pltpu.make_async_copy(
            zr[b], part_out.at[pl.ds(c * NPAD, ZROWS)], osem[b]).wait()


def _aggregate(src, dst, y):
    k = pl.kernel(
        _agg_kernel,
        out_type=jax.ShapeDtypeStruct((NC * NPAD, F), jnp.float32),
        mesh=plsc.VectorSubcoreMesh(core_axis_name="c",
                                    subcore_axis_name="s"),
        scratch_types=(
            [pltpu.VMEM((C4, F), jnp.float32)] * 6
            + [pltpu.SemaphoreType.DMA] * 12
            + [pltpu.VMEM((C4,), jnp.int32)] * 24
            + [pltpu.SemaphoreType.DMA] * 12
            + [pltpu.VMEM((ZROWS, F), jnp.float32)] * 2
            + [pltpu.SemaphoreType.DMA] * 2
            + [pltpu.VMEM_SHARED((NPAD, F), jnp.float32)]
        ),
    )
    return k(src, dst, y)


def _head_kernel(p_ref, y_ref, dinv_ref, lw_ref, lb_ref, out_ref):
    t = p_ref[0] + p_ref[1] + y_ref[...]
    h = jnp.maximum(t, 0.0) * dinv_ref[...]
    out_ref[...] = lax.dot_general(
        h, lw_ref[...], (((1,), (1,)), ((), ())),
        preferred_element_type=jnp.float32) + lb_ref[...]


def _head(parts, y, dinv, lin_w, lin_b):
    blk = 2048
    grid = NPAD // blk
    return pl.pallas_call(
        _head_kernel,
        grid=(grid,),
        out_shape=jax.ShapeDtypeStruct((N, L), jnp.float32),
        in_specs=[
            pl.BlockSpec((NC, blk, F), lambda i: (0, i, 0)),
            pl.BlockSpec((blk, F), lambda i: (i, 0)),
            pl.BlockSpec((blk, 1), lambda i: (i, 0)),
            pl.BlockSpec((L, F), lambda i: (0, 0)),
            pl.BlockSpec((1, L), lambda i: (0, 0)),
        ],
        out_specs=pl.BlockSpec((blk, L), lambda i: (i, 0)),
    )(parts, y, dinv, lin_w, lin_b.reshape(1, L))


def kernel(x, edge_index, pool_p, W0, gru_w_ih, gru_w_hh, gru_b_ih,
           gru_b_hh, lin_w, lin_b):
    src = edge_index[0]
    dst = edge_index[1]
    s_col = _score(x, pool_p)
    score2d = jnp.pad(s_col[:, 0], (0, NPAD - N),
                      constant_values=NEG).reshape(SROWS, 128)
    deg2 = _degrees(dst)
    d0 = deg2[:N].reshape(N, 1)
    d1 = deg2[NPAD:NPAD + N].reshape(N, 1)
    y, dinv = _evolve_y(score2d, x, W0, gru_w_ih, gru_w_hh, gru_b_ih,
                        gru_b_hh, d0, d1)
    parts = _aggregate(src, dst, y).reshape(NC, NPAD, F)
    return _head(parts, y, dinv, lin_w, lin_b)
